# Initial kernel scaffold; baseline (speedup 1.0000x reference)
import jax
import jax.numpy as jnp
from jax.experimental import pallas as pl


def kernel(x, pos_emb):
    raise NotImplementedError("write your pallas kernel here")


# speedup vs baseline: 3.0124x; 3.0124x over previous
import jax
import jax.numpy as jnp
from jax.experimental import pallas as pl


def _copy_block(src_ref, out_ref):
    out_ref[...] = src_ref[...]


def kernel(x, pos_emb):
    seq_len = x.shape[1]
    dim = pos_emb.shape[1]
    block = 512
    grid = (seq_len // block,)
    return pl.pallas_call(
        _copy_block,
        grid=grid,
        in_specs=[pl.BlockSpec((block, dim), lambda i: (i, 0))],
        out_specs=pl.BlockSpec((block, dim), lambda i: (i, 0)),
        out_shape=jax.ShapeDtypeStruct((seq_len, dim), pos_emb.dtype),
    )(pos_emb)
